# baseline (speedup 1.0000x reference)
import functools

import jax
import jax.numpy as jnp
from jax import lax
from jax.experimental import pallas as pl
from jax.experimental.pallas import tpu as pltpu
from jax.experimental.pallas import tpu_sc as plsc

N_NODES = 10000
D = 128
NC, NS, LANES = 2, 16, 16
NW = NC * NS
K = 128
NRB = 3
NMS = 4
UNROLL = 12

_mesh = plsc.VectorSubcoreMesh(
    core_axis_name="c", subcore_axis_name="s", num_cores=NC, num_subcores=NS)


N_PAD = 10240


def _deg_body(dst_hbm, ew_hbm, out_hbm, dst_v, ew_v, deg_v):
    epw = dst_v.shape[0]
    c = lax.axis_index("c")
    s = lax.axis_index("s")
    wid = c * NS + s
    zeros = jnp.zeros((LANES,), jnp.float32)

    def zbody(i, carry):
        deg_v[pl.ds(i * LANES, LANES)] = zeros
        return carry

    lax.fori_loop(0, N_PAD // LANES, zbody, 0)
    pltpu.sync_copy(dst_hbm.at[wid], dst_v)
    pltpu.sync_copy(ew_hbm.at[wid], ew_v)

    def body(g, carry):
        idx = dst_v[pl.ds(g * LANES, LANES)]
        val = ew_v[pl.ds(g * LANES, LANES)]
        plsc.addupdate_scatter(deg_v, [idx], val)
        return carry

    lax.fori_loop(0, epw // LANES, body, 0)
    pltpu.sync_copy(deg_v, out_hbm.at[wid])


def _make_deg_call(epw):
    return pl.kernel(
        _deg_body,
        out_type=jax.ShapeDtypeStruct((NW, N_PAD), jnp.float32),
        mesh=_mesh,
        scratch_types=[
            pltpu.VMEM((epw,), jnp.int32),
            pltpu.VMEM((epw,), jnp.float32),
            pltpu.VMEM((N_PAD,), jnp.float32),
        ],
        compiler_params=pltpu.CompilerParams(needs_layout_passes=False),
    )


def _agg_body(ch_split, hp, src2, dst2, ew2, out_hbm, sidx_v, didx_v, ew_v,
              rows_v, acc_sh, m0, m1, m2, m3, g0, g1, g2, s0, s1, s2):
    ch0, ch1 = ch_split
    n_acc = out_hbm.shape[1]
    msems = (m0, m1, m2, m3)
    gsems = (g0, g1, g2)
    ssems = (s0, s1, s2)
    c = lax.axis_index("c")
    s = lax.axis_index("s")
    ch_n = jnp.where(c == 0, ch0, ch1)
    base = jnp.where(c == 0, s * ch0, NS * ch0 + s * ch1)
    rpt = 632

    def meta_start(ch, sl):
        row = base + ch
        pltpu.async_copy(src2.at[row], sidx_v.at[pl.ds(sl * K, K)],
                         msems[sl])
        pltpu.async_copy(dst2.at[row], didx_v.at[sl], msems[sl])
        pltpu.async_copy(ew2.at[row], ew_v.at[pl.ds(sl * K, K)],
                         msems[sl])

    def meta_wait(ch, sl):
        row = base + ch
        pltpu.make_async_copy(src2.at[row],
                              sidx_v.at[pl.ds(sl * K, K)], msems[sl]).wait()
        pltpu.make_async_copy(dst2.at[row], didx_v.at[sl],
                              msems[sl]).wait()
        pltpu.make_async_copy(ew2.at[row], ew_v.at[pl.ds(sl * K, K)],
                              msems[sl]).wait()

    def gather_start(b, sl):
        pltpu.async_copy(hp.at[sidx_v.at[pl.ds(sl * K, K)]], rows_v.at[b],
                         gsems[b])

    def gather_wait(b, sl):
        pltpu.make_async_copy(hp.at[sidx_v.at[pl.ds(sl * K, K)]],
                              rows_v.at[b], gsems[b]).wait()

    def scatter_start(b, sl):
        pltpu.async_copy(rows_v.at[b], acc_sh.at[didx_v.at[sl]],
                         ssems[b], add=True)

    def scatter_wait(b, sl):
        pltpu.make_async_copy(rows_v.at[b], acc_sh.at[didx_v.at[sl]],
                              ssems[b]).wait()

    zeros = jnp.zeros((LANES,), jnp.float32)

    def zbody(i, carry):
        for v in range(D // LANES):
            rows_v[0, i, pl.ds(v * LANES, LANES)] = zeros
        return carry

    lax.fori_loop(0, K, zbody, 0)

    def acc_blocks(tile):
        lo = tile * rpt
        hi = min((tile + 1) * rpt, n_acc)
        out, off = [], lo
        while off < hi:
            sz = min(K, hi - off)
            out.append((off, sz))
            off += sz
        return out

    def zero_tile(tile):
        def f():
            for off, sz in acc_blocks(tile):
                pltpu.sync_copy(rows_v.at[0, pl.ds(0, sz)],
                                acc_sh.at[pl.ds(off, sz)])
        return f

    for tile in range(NS):
        pl.when(s == tile)(zero_tile(tile))
    plsc.subcore_barrier()

    meta_start(0, 0)
    meta_start(1, 1)
    meta_start(2, 2)
    meta_wait(0, 0)
    gather_start(0, 0)
    meta_wait(1, 1)
    gather_start(1, 1)

    def make_process(ch, pos):
        sl = pos % NMS
        b = pos % NRB
        sl2 = (pos + 2) % NMS
        b2 = (pos + 2) % NRB
        slm1 = (pos - 1) % NMS
        bm1 = (pos - 1) % NRB
        sl3 = (pos + 3) % NMS
        static = isinstance(ch, int)

        gather_wait(b, sl)

        def wgroup(g, carry2, _b=b, _sl=sl):
            ewv = ew_v[pl.ds(_sl * K + g * LANES, LANES)]
            for l in range(LANES):
                w = lax.gather(
                    ewv, jnp.full((LANES, 1), l, jnp.int32),
                    lax.GatherDimensionNumbers(
                        offset_dims=(), collapsed_slice_dims=(0,),
                        start_index_map=(0,)),
                    slice_sizes=(1,),
                    mode=lax.GatherScatterMode.PROMISE_IN_BOUNDS)
                r = g * LANES + l
                for v in range(D // LANES):
                    sli = pl.ds(v * LANES, LANES)
                    rows_v[_b, r, sli] = rows_v[_b, r, sli] * w
            return carry2

        lax.fori_loop(0, K // LANES, wgroup, 0)

        def prep2():
            meta_wait(ch + 2, sl2)
            if not (static and ch == 0):
                scatter_wait(bm1, slm1)
            gather_start(b2, sl2)

        def prep3():
            meta_start(ch + 3, sl3)

        pl.when(ch + 2 < ch_n)(prep2)
        scatter_start(b, sl)
        pl.when(ch + 3 < ch_n)(prep3)

    for pos in range(UNROLL):
        make_process(pos, pos)

    def outer(o, carry):
        obase = o * UNROLL
        for pos in range(UNROLL):
            make_process(obase + pos, pos)
        return carry

    lax.fori_loop(1, ch_n // UNROLL, outer, 0)
    for back in (3, 2, 1):
        scatter_wait((-back) % NRB, (-back) % NMS)
    plsc.subcore_barrier()

    def flush_tile(tile):
        def f():
            for off, sz in acc_blocks(tile):
                pltpu.sync_copy(acc_sh.at[pl.ds(off, sz)],
                                rows_v.at[0, pl.ds(0, sz)])
                pltpu.sync_copy(rows_v.at[0, pl.ds(0, sz)],
                                out_hbm.at[c, pl.ds(off, sz)])
        return f

    for tile in range(NS):
        pl.when(s == tile)(flush_tile(tile))


def _make_agg_call(n, ch_split):
    return pl.kernel(
        functools.partial(_agg_body, ch_split),
        out_type=jax.ShapeDtypeStruct((NC, n, D), jnp.float32),
        mesh=_mesh,
        scratch_types=[
            pltpu.VMEM((NMS * K,), jnp.int32),
            pltpu.VMEM((NMS, K), jnp.int32),
            pltpu.VMEM((NMS * K,), jnp.float32),
            pltpu.VMEM((NRB, K, D), jnp.float32),
            pltpu.VMEM_SHARED((n, D), jnp.float32),
        ] + [pltpu.SemaphoreType.DMA] * (NMS + 2 * NRB),
        compiler_params=pltpu.CompilerParams(needs_layout_passes=False),
    )


_BLKN = 2048
_R = 2000


def _dis_body(parts_ref, dis_ref):
    deg = jnp.sum(parts_ref[...], axis=0) + 1.0
    dis = lax.rsqrt(deg)
    dis_ref[...] = jnp.broadcast_to(dis[:, None], (_BLKN, D))


def _h1_body(x_ref, w_ref, dis_ref, out_ref):
    h = jnp.dot(x_ref[...], w_ref[...], preferred_element_type=jnp.float32)
    out_ref[...] = h * dis_ref[...]


def _mid_body(a0_ref, a1_ref, hp_ref, dis_ref, b1_ref, w2_ref, out_ref):
    t = (a0_ref[...] + a1_ref[...] + hp_ref[...]) * dis_ref[...] + b1_ref[...]
    t = jnp.maximum(t, 0.0)
    h = jnp.dot(t, w2_ref[...], preferred_element_type=jnp.float32)
    out_ref[...] = h * dis_ref[...]


def _out_body(a0_ref, a1_ref, hp_ref, dis_ref, b2_ref, out_ref):
    out_ref[...] = ((a0_ref[...] + a1_ref[...] + hp_ref[...]) * dis_ref[...]
                    + b2_ref[...])


def _row_spec(r):
    return pl.BlockSpec((r, D), lambda i: (i, 0))


def _full_spec(shape):
    return pl.BlockSpec(shape, lambda i: tuple(0 for _ in shape))


_CH_SPLIT = (24, 144)


@jax.jit
def kernel(x, edge_index, edge_attr, W1, b1, W2, b2):
    n, d = x.shape
    e = edge_index.shape[1]
    ch0, ch1 = _CH_SPLIT
    tot_ch = NS * (ch0 + ch1)
    e_pad = tot_ch * K
    assert e_pad >= e and ch0 % UNROLL == 0 and ch1 % UNROLL == 0
    epw = e_pad // NW

    src = edge_index[0].astype(jnp.int32)
    dst = edge_index[1].astype(jnp.int32)
    ew = edge_attr.astype(jnp.float32)
    pad = e_pad - e
    src3 = jnp.concatenate([src, jnp.zeros((pad,), jnp.int32)]
                           ).reshape(tot_ch, K)
    dst3 = jnp.concatenate([dst, jnp.zeros((pad,), jnp.int32)]
                           ).reshape(tot_ch, K)
    ew3 = jnp.concatenate([ew, jnp.zeros((pad,), jnp.float32)]
                          ).reshape(tot_ch, K)

    deg_parts = _make_deg_call(epw)(
        dst3.reshape(NW, epw), ew3.reshape(NW, epw))

    dis_b = pl.pallas_call(
        _dis_body,
        grid=(N_PAD // _BLKN,),
        in_specs=[pl.BlockSpec((NW, _BLKN), lambda i: (0, i))],
        out_specs=_row_spec(_BLKN),
        out_shape=jax.ShapeDtypeStruct((N_PAD, D), jnp.float32),
    )(deg_parts)

    h1p = pl.pallas_call(
        _h1_body,
        grid=(n // _R,),
        in_specs=[_row_spec(_R), _full_spec((D, D)), _row_spec(_R)],
        out_specs=_row_spec(_R),
        out_shape=jax.ShapeDtypeStruct((n, D), jnp.float32),
    )(x, W1, dis_b)

    agg_call = _make_agg_call(n, _CH_SPLIT)
    agg1 = agg_call(h1p, src3, dst3, ew3)

    h2p = pl.pallas_call(
        _mid_body,
        grid=(n // _R,),
        in_specs=[_row_spec(_R), _row_spec(_R), _row_spec(_R), _row_spec(_R),
                  _full_spec((1, D)), _full_spec((D, D))],
        out_specs=_row_spec(_R),
        out_shape=jax.ShapeDtypeStruct((n, D), jnp.float32),
    )(agg1[0], agg1[1], h1p, dis_b, b1.reshape(1, D), W2)

    agg2 = agg_call(h2p, src3, dst3, ew3)

    out = pl.pallas_call(
        _out_body,
        grid=(n // _R,),
        in_specs=[_row_spec(_R), _row_spec(_R), _row_spec(_R), _row_spec(_R),
                  _full_spec((1, D))],
        out_specs=_row_spec(_R),
        out_shape=jax.ShapeDtypeStruct((n, D), jnp.float32),
    )(agg2[0], agg2[1], h2p, dis_b, b2.reshape(1, D))
    return out


# speedup vs baseline: 1.0348x; 1.0348x over previous
import functools

import jax
import jax.numpy as jnp
from jax import lax
from jax.experimental import pallas as pl
from jax.experimental.pallas import tpu as pltpu
from jax.experimental.pallas import tpu_sc as plsc

N_NODES = 10000
D = 128
NC, NS, LANES = 2, 16, 16
NW = NC * NS
K = 128
NRB = 3
NMS = 4
UNROLL = 12

_mesh = plsc.VectorSubcoreMesh(
    core_axis_name="c", subcore_axis_name="s", num_cores=NC, num_subcores=NS)


N_PAD = 10240


def _deg_body(dst_hbm, ew_hbm, out_hbm, dst_v, ew_v, deg_v):
    epw = dst_v.shape[0]
    c = lax.axis_index("c")
    s = lax.axis_index("s")
    wid = c * NS + s
    zeros = jnp.zeros((LANES,), jnp.float32)

    def zbody(i, carry):
        deg_v[pl.ds(i * LANES, LANES)] = zeros
        return carry

    lax.fori_loop(0, N_PAD // LANES, zbody, 0)
    pltpu.sync_copy(dst_hbm.at[wid], dst_v)
    pltpu.sync_copy(ew_hbm.at[wid], ew_v)

    def body(g, carry):
        idx = dst_v[pl.ds(g * LANES, LANES)]
        val = ew_v[pl.ds(g * LANES, LANES)]
        plsc.addupdate_scatter(deg_v, [idx], val)
        return carry

    lax.fori_loop(0, epw // LANES, body, 0)
    pltpu.sync_copy(deg_v, out_hbm.at[wid])


def _make_deg_call(epw):
    return pl.kernel(
        _deg_body,
        out_type=jax.ShapeDtypeStruct((NW, N_PAD), jnp.float32),
        mesh=_mesh,
        scratch_types=[
            pltpu.VMEM((epw,), jnp.int32),
            pltpu.VMEM((epw,), jnp.float32),
            pltpu.VMEM((N_PAD,), jnp.float32),
        ],
        compiler_params=pltpu.CompilerParams(needs_layout_passes=False),
    )


def _agg_body(ch_split, hp, src2, dst2, ew2, out_hbm, sidx_v, didx_v, ew_v,
              rows_v, acc_sh, m0, m1, m2, m3, g0, g1, g2, s0, s1, s2):
    ch0, ch1 = ch_split
    n_acc = out_hbm.shape[1]
    msems = (m0, m1, m2, m3)
    gsems = (g0, g1, g2)
    ssems = (s0, s1, s2)
    c = lax.axis_index("c")
    s = lax.axis_index("s")
    ch_n = jnp.where(c == 0, ch0, ch1)
    base = jnp.where(c == 0, s * ch0, NS * ch0 + s * ch1)
    rpt = 632

    def meta_start(ch, sl):
        row = base + ch
        pltpu.async_copy(src2.at[row], sidx_v.at[pl.ds(sl * K, K)],
                         msems[sl])
        pltpu.async_copy(dst2.at[row], didx_v.at[sl], msems[sl])
        pltpu.async_copy(ew2.at[row], ew_v.at[pl.ds(sl * K, K)],
                         msems[sl])

    def meta_wait(ch, sl):
        row = base + ch
        pltpu.make_async_copy(src2.at[row],
                              sidx_v.at[pl.ds(sl * K, K)], msems[sl]).wait()
        pltpu.make_async_copy(dst2.at[row], didx_v.at[sl],
                              msems[sl]).wait()
        pltpu.make_async_copy(ew2.at[row], ew_v.at[pl.ds(sl * K, K)],
                              msems[sl]).wait()

    def gather_start(b, sl):
        pltpu.async_copy(hp.at[sidx_v.at[pl.ds(sl * K, K)]], rows_v.at[b],
                         gsems[b])

    def gather_wait(b, sl):
        pltpu.make_async_copy(hp.at[sidx_v.at[pl.ds(sl * K, K)]],
                              rows_v.at[b], gsems[b]).wait()

    def scatter_start(b, sl):
        pltpu.async_copy(rows_v.at[b], acc_sh.at[didx_v.at[sl]],
                         ssems[b], add=True)

    def scatter_wait(b, sl):
        pltpu.make_async_copy(rows_v.at[b], acc_sh.at[didx_v.at[sl]],
                              ssems[b]).wait()

    zeros = jnp.zeros((LANES,), jnp.float32)

    def zbody(i, carry):
        for v in range(D // LANES):
            rows_v[0, i, pl.ds(v * LANES, LANES)] = zeros
        return carry

    lax.fori_loop(0, K, zbody, 0)

    _blk = [(0, K), (K, K), (2 * K, K), (3 * K, K), (4 * K, rpt - 4 * K)]

    def acc_off(boff, sz):
        off = jnp.minimum(s * rpt + boff, n_acc - sz)
        return pl.multiple_of(off, 8)

    for boff, sz in _blk:
        pltpu.sync_copy(rows_v.at[0, pl.ds(0, sz)],
                        acc_sh.at[pl.ds(acc_off(boff, sz), sz)])
    plsc.subcore_barrier()

    meta_start(0, 0)
    meta_start(1, 1)
    meta_start(2, 2)
    meta_wait(0, 0)
    gather_start(0, 0)
    meta_wait(1, 1)
    gather_start(1, 1)

    def make_process(ch, pos):
        sl = pos % NMS
        b = pos % NRB
        sl2 = (pos + 2) % NMS
        b2 = (pos + 2) % NRB
        slm1 = (pos - 1) % NMS
        bm1 = (pos - 1) % NRB
        sl3 = (pos + 3) % NMS
        static = isinstance(ch, int)

        gather_wait(b, sl)

        def wgroup(g, carry2, _b=b, _sl=sl):
            ewv = ew_v[pl.ds(_sl * K + g * LANES, LANES)]
            for l in range(LANES):
                w = lax.gather(
                    ewv, jnp.full((LANES, 1), l, jnp.int32),
                    lax.GatherDimensionNumbers(
                        offset_dims=(), collapsed_slice_dims=(0,),
                        start_index_map=(0,)),
                    slice_sizes=(1,),
                    mode=lax.GatherScatterMode.PROMISE_IN_BOUNDS)
                r = g * LANES + l
                for v in range(D // LANES):
                    sli = pl.ds(v * LANES, LANES)
                    rows_v[_b, r, sli] = rows_v[_b, r, sli] * w
            return carry2

        lax.fori_loop(0, K // LANES, wgroup, 0)

        def prep2():
            meta_wait(ch + 2, sl2)
            if not (static and ch == 0):
                scatter_wait(bm1, slm1)
            gather_start(b2, sl2)

        def prep3():
            meta_start(ch + 3, sl3)

        pl.when(ch + 2 < ch_n)(prep2)
        scatter_start(b, sl)
        pl.when(ch + 3 < ch_n)(prep3)

    for pos in range(UNROLL):
        make_process(pos, pos)

    def outer(o, carry):
        obase = o * UNROLL
        for pos in range(UNROLL):
            make_process(obase + pos, pos)
        return carry

    lax.fori_loop(1, ch_n // UNROLL, outer, 0)
    for back in (3, 2, 1):
        scatter_wait((-back) % NRB, (-back) % NMS)
    plsc.subcore_barrier()

    for boff, sz in _blk:
        off = acc_off(boff, sz)
        pltpu.sync_copy(acc_sh.at[pl.ds(off, sz)],
                        rows_v.at[0, pl.ds(0, sz)])
        pltpu.sync_copy(rows_v.at[0, pl.ds(0, sz)],
                        out_hbm.at[c, pl.ds(off, sz)])


def _make_agg_call(n, ch_split):
    return pl.kernel(
        functools.partial(_agg_body, ch_split),
        out_type=jax.ShapeDtypeStruct((NC, n, D), jnp.float32),
        mesh=_mesh,
        scratch_types=[
            pltpu.VMEM((NMS * K,), jnp.int32),
            pltpu.VMEM((NMS, K), jnp.int32),
            pltpu.VMEM((NMS * K,), jnp.float32),
            pltpu.VMEM((NRB, K, D), jnp.float32),
            pltpu.VMEM_SHARED((n, D), jnp.float32),
        ] + [pltpu.SemaphoreType.DMA] * (NMS + 2 * NRB),
        compiler_params=pltpu.CompilerParams(needs_layout_passes=False),
    )


_BLKN = 2048
_R = 2000


def _dis_body(parts_ref, dis_ref):
    deg = jnp.sum(parts_ref[...], axis=0) + 1.0
    dis = lax.rsqrt(deg)
    dis_ref[...] = jnp.broadcast_to(dis[:, None], (_BLKN, D))


def _h1_body(x_ref, w_ref, dis_ref, out_ref):
    h = jnp.dot(x_ref[...], w_ref[...], preferred_element_type=jnp.float32)
    out_ref[...] = h * dis_ref[...]


def _mid_body(a0_ref, a1_ref, hp_ref, dis_ref, b1_ref, w2_ref, out_ref):
    t = (a0_ref[...] + a1_ref[...] + hp_ref[...]) * dis_ref[...] + b1_ref[...]
    t = jnp.maximum(t, 0.0)
    h = jnp.dot(t, w2_ref[...], preferred_element_type=jnp.float32)
    out_ref[...] = h * dis_ref[...]


def _out_body(a0_ref, a1_ref, hp_ref, dis_ref, b2_ref, out_ref):
    out_ref[...] = ((a0_ref[...] + a1_ref[...] + hp_ref[...]) * dis_ref[...]
                    + b2_ref[...])


def _row_spec(r):
    return pl.BlockSpec((r, D), lambda i: (i, 0))


def _full_spec(shape):
    return pl.BlockSpec(shape, lambda i: tuple(0 for _ in shape))


_CH_SPLIT = (84, 84)


@jax.jit
def kernel(x, edge_index, edge_attr, W1, b1, W2, b2):
    n, d = x.shape
    e = edge_index.shape[1]
    ch0, ch1 = _CH_SPLIT
    tot_ch = NS * (ch0 + ch1)
    e_pad = tot_ch * K
    assert e_pad >= e and ch0 % UNROLL == 0 and ch1 % UNROLL == 0
    epw = e_pad // NW

    src = edge_index[0].astype(jnp.int32)
    dst = edge_index[1].astype(jnp.int32)
    ew = edge_attr.astype(jnp.float32)
    pad = e_pad - e
    src3 = jnp.concatenate([src, jnp.zeros((pad,), jnp.int32)]
                           ).reshape(tot_ch, K)
    dst3 = jnp.concatenate([dst, jnp.zeros((pad,), jnp.int32)]
                           ).reshape(tot_ch, K)
    ew3 = jnp.concatenate([ew, jnp.zeros((pad,), jnp.float32)]
                          ).reshape(tot_ch, K)

    deg_parts = _make_deg_call(epw)(
        dst3.reshape(NW, epw), ew3.reshape(NW, epw))

    dis_b = pl.pallas_call(
        _dis_body,
        grid=(N_PAD // _BLKN,),
        in_specs=[pl.BlockSpec((NW, _BLKN), lambda i: (0, i))],
        out_specs=_row_spec(_BLKN),
        out_shape=jax.ShapeDtypeStruct((N_PAD, D), jnp.float32),
    )(deg_parts)

    h1p = pl.pallas_call(
        _h1_body,
        grid=(n // _R,),
        in_specs=[_row_spec(_R), _full_spec((D, D)), _row_spec(_R)],
        out_specs=_row_spec(_R),
        out_shape=jax.ShapeDtypeStruct((n, D), jnp.float32),
    )(x, W1, dis_b)

    agg_call = _make_agg_call(n, _CH_SPLIT)
    agg1 = agg_call(h1p, src3, dst3, ew3)

    h2p = pl.pallas_call(
        _mid_body,
        grid=(n // _R,),
        in_specs=[_row_spec(_R), _row_spec(_R), _row_spec(_R), _row_spec(_R),
                  _full_spec((1, D)), _full_spec((D, D))],
        out_specs=_row_spec(_R),
        out_shape=jax.ShapeDtypeStruct((n, D), jnp.float32),
    )(agg1[0], agg1[1], h1p, dis_b, b1.reshape(1, D), W2)

    agg2 = agg_call(h2p, src3, dst3, ew3)

    out = pl.pallas_call(
        _out_body,
        grid=(n // _R,),
        in_specs=[_row_spec(_R), _row_spec(_R), _row_spec(_R), _row_spec(_R),
                  _full_spec((1, D))],
        out_specs=_row_spec(_R),
        out_shape=jax.ShapeDtypeStruct((n, D), jnp.float32),
    )(agg2[0], agg2[1], h2p, dis_b, b2.reshape(1, D))
    return out


# speedup vs baseline: 1.0592x; 1.0236x over previous
import functools

import jax
import jax.numpy as jnp
from jax import lax
from jax.experimental import pallas as pl
from jax.experimental.pallas import tpu as pltpu
from jax.experimental.pallas import tpu_sc as plsc

N_NODES = 10000
D = 128
NC, NS, LANES = 2, 16, 16
NW = NC * NS
K = 128
NRB = 3
NMS = 4
UNROLL = 12

_mesh = plsc.VectorSubcoreMesh(
    core_axis_name="c", subcore_axis_name="s", num_cores=NC, num_subcores=NS)


N_PAD = 10240


def _deg_body(dst_hbm, ew_hbm, out_hbm, dst_v, ew_v, deg_v):
    epw = dst_v.shape[0]
    c = lax.axis_index("c")
    s = lax.axis_index("s")
    wid = c * NS + s
    zeros = jnp.zeros((LANES,), jnp.float32)

    def zbody(i, carry):
        deg_v[pl.ds(i * LANES, LANES)] = zeros
        return carry

    lax.fori_loop(0, N_PAD // LANES, zbody, 0)
    pltpu.sync_copy(dst_hbm.at[wid], dst_v)
    pltpu.sync_copy(ew_hbm.at[wid], ew_v)

    def body(g, carry):
        idx = dst_v[pl.ds(g * LANES, LANES)]
        val = ew_v[pl.ds(g * LANES, LANES)]
        plsc.addupdate_scatter(deg_v, [idx], val)
        return carry

    lax.fori_loop(0, epw // LANES, body, 0)
    pltpu.sync_copy(deg_v, out_hbm.at[wid])


def _make_deg_call(epw):
    return pl.kernel(
        _deg_body,
        out_type=jax.ShapeDtypeStruct((NW, N_PAD), jnp.float32),
        mesh=_mesh,
        scratch_types=[
            pltpu.VMEM((epw,), jnp.int32),
            pltpu.VMEM((epw,), jnp.float32),
            pltpu.VMEM((N_PAD,), jnp.float32),
        ],
        compiler_params=pltpu.CompilerParams(needs_layout_passes=False),
    )


def _agg_body(ch_split, hp, src2, dst2, ew2, out_hbm, sidx_v, didx_v, ew_v,
              rows_v, acc_sh, msem, gsem, ssem):
    ch0, ch1 = ch_split
    n_acc = out_hbm.shape[1]
    c = lax.axis_index("c")
    s = lax.axis_index("s")
    ch_n = jnp.where(c == 0, ch0, ch1)
    base = jnp.where(c == 0, s * ch0, NS * ch0 + s * ch1)
    rpt = 632

    def meta_start(ch, sl):
        row = base + ch
        pltpu.async_copy(src2.at[row], sidx_v.at[pl.ds(sl * K, K)],
                         msem.at[sl])
        pltpu.async_copy(dst2.at[row], didx_v.at[sl], msem.at[sl])
        pltpu.async_copy(ew2.at[row], ew_v.at[pl.ds(sl * K, K)],
                         msem.at[sl])

    def meta_wait(ch, sl):
        row = base + ch
        pltpu.make_async_copy(src2.at[row],
                              sidx_v.at[pl.ds(sl * K, K)],
                              msem.at[sl]).wait()
        pltpu.make_async_copy(dst2.at[row], didx_v.at[sl],
                              msem.at[sl]).wait()
        pltpu.make_async_copy(ew2.at[row], ew_v.at[pl.ds(sl * K, K)],
                              msem.at[sl]).wait()

    def gather_start(b, sl):
        pltpu.async_copy(hp.at[sidx_v.at[pl.ds(sl * K, K)]], rows_v.at[b],
                         gsem.at[b])

    def gather_wait(b, sl):
        pltpu.make_async_copy(hp.at[sidx_v.at[pl.ds(sl * K, K)]],
                              rows_v.at[b], gsem.at[b]).wait()

    def scatter_start(b, sl):
        pltpu.async_copy(rows_v.at[b], acc_sh.at[didx_v.at[sl]],
                         ssem.at[b], add=True)

    def scatter_wait(b, sl):
        pltpu.make_async_copy(rows_v.at[b], acc_sh.at[didx_v.at[sl]],
                              ssem.at[b]).wait()

    zeros = jnp.zeros((LANES,), jnp.float32)

    def zbody(i, carry):
        for v in range(D // LANES):
            rows_v[0, i, pl.ds(v * LANES, LANES)] = zeros
        return carry

    lax.fori_loop(0, K, zbody, 0)

    _blk = [(0, K), (K, K), (2 * K, K), (3 * K, K), (4 * K, rpt - 4 * K)]

    def acc_off(boff, sz):
        off = jnp.minimum(s * rpt + boff, n_acc - sz)
        return pl.multiple_of(off, 8)

    for boff, sz in _blk:
        pltpu.sync_copy(rows_v.at[0, pl.ds(0, sz)],
                        acc_sh.at[pl.ds(acc_off(boff, sz), sz)])
    plsc.subcore_barrier()

    meta_start(0, 0)
    meta_start(1, 1)
    meta_start(2, 2)
    meta_wait(0, 0)
    gather_start(0, 0)
    meta_wait(1, 1)
    gather_start(1, 1)

    def process(ch, carry):
        sl = lax.rem(ch, NMS)
        b = lax.rem(ch, NRB)
        sl2 = lax.rem(ch + 2, NMS)
        b2 = lax.rem(ch + 2, NRB)
        slm1 = lax.rem(ch + NMS - 1, NMS)
        bm1 = lax.rem(ch + NRB - 1, NRB)
        sl3 = lax.rem(ch + 3, NMS)

        gather_wait(b, sl)

        def wgroup(g, carry2):
            ewv = ew_v[pl.ds(sl * K + g * LANES, LANES)]
            for l in range(LANES):
                w = lax.gather(
                    ewv, jnp.full((LANES, 1), l, jnp.int32),
                    lax.GatherDimensionNumbers(
                        offset_dims=(), collapsed_slice_dims=(0,),
                        start_index_map=(0,)),
                    slice_sizes=(1,),
                    mode=lax.GatherScatterMode.PROMISE_IN_BOUNDS)
                r = g * LANES + l
                for v in range(D // LANES):
                    sli = pl.ds(v * LANES, LANES)
                    rows_v[b, r, sli] = rows_v[b, r, sli] * w
            return carry2

        lax.fori_loop(0, K // LANES, wgroup, 0)

        def prep2():
            meta_wait(ch + 2, sl2)
            pl.when(ch >= 1)(lambda: scatter_wait(bm1, slm1))
            gather_start(b2, sl2)

        pl.when(ch + 2 < ch_n)(prep2)
        scatter_start(b, sl)
        pl.when(ch + 3 < ch_n)(lambda: meta_start(ch + 3, sl3))
        return carry

    lax.fori_loop(0, ch_n, process, 0)
    for back in (3, 2, 1):
        chb = ch_n - back
        scatter_wait(lax.rem(chb, NRB), lax.rem(chb, NMS))
    plsc.subcore_barrier()

    for boff, sz in _blk:
        off = acc_off(boff, sz)
        pltpu.sync_copy(acc_sh.at[pl.ds(off, sz)],
                        rows_v.at[0, pl.ds(0, sz)])
        pltpu.sync_copy(rows_v.at[0, pl.ds(0, sz)],
                        out_hbm.at[c, pl.ds(off, sz)])


def _make_agg_call(n, ch_split):
    return pl.kernel(
        functools.partial(_agg_body, ch_split),
        out_type=jax.ShapeDtypeStruct((NC, n, D), jnp.float32),
        mesh=_mesh,
        scratch_types=[
            pltpu.VMEM((NMS * K,), jnp.int32),
            pltpu.VMEM((NMS, K), jnp.int32),
            pltpu.VMEM((NMS * K,), jnp.float32),
            pltpu.VMEM((NRB, K, D), jnp.float32),
            pltpu.VMEM_SHARED((n, D), jnp.float32),
            pltpu.SemaphoreType.DMA((NMS,)),
            pltpu.SemaphoreType.DMA((NRB,)),
            pltpu.SemaphoreType.DMA((NRB,)),
        ],
        compiler_params=pltpu.CompilerParams(needs_layout_passes=False),
    )


_BLKN = 2048
_R = 2000


def _dis_body(parts_ref, dis_ref):
    deg = jnp.sum(parts_ref[...], axis=0) + 1.0
    dis = lax.rsqrt(deg)
    dis_ref[...] = jnp.broadcast_to(dis[:, None], (_BLKN, D))


def _h1_body(x_ref, w_ref, dis_ref, out_ref):
    h = jnp.dot(x_ref[...], w_ref[...], preferred_element_type=jnp.float32)
    out_ref[...] = h * dis_ref[...]


def _mid_body(a0_ref, a1_ref, hp_ref, dis_ref, b1_ref, w2_ref, out_ref):
    t = (a0_ref[...] + a1_ref[...] + hp_ref[...]) * dis_ref[...] + b1_ref[...]
    t = jnp.maximum(t, 0.0)
    h = jnp.dot(t, w2_ref[...], preferred_element_type=jnp.float32)
    out_ref[...] = h * dis_ref[...]


def _out_body(a0_ref, a1_ref, hp_ref, dis_ref, b2_ref, out_ref):
    out_ref[...] = ((a0_ref[...] + a1_ref[...] + hp_ref[...]) * dis_ref[...]
                    + b2_ref[...])


def _row_spec(r):
    return pl.BlockSpec((r, D), lambda i: (i, 0))


def _full_spec(shape):
    return pl.BlockSpec(shape, lambda i: tuple(0 for _ in shape))


_CH_SPLIT = (84, 84)


@jax.jit
def kernel(x, edge_index, edge_attr, W1, b1, W2, b2):
    n, d = x.shape
    e = edge_index.shape[1]
    ch0, ch1 = _CH_SPLIT
    tot_ch = NS * (ch0 + ch1)
    e_pad = tot_ch * K
    assert e_pad >= e and min(ch0, ch1) >= 4
    epw = e_pad // NW

    src = edge_index[0].astype(jnp.int32)
    dst = edge_index[1].astype(jnp.int32)
    ew = edge_attr.astype(jnp.float32)
    pad = e_pad - e
    src3 = jnp.concatenate([src, jnp.zeros((pad,), jnp.int32)]
                           ).reshape(tot_ch, K)
    dst3 = jnp.concatenate([dst, jnp.zeros((pad,), jnp.int32)]
                           ).reshape(tot_ch, K)
    ew3 = jnp.concatenate([ew, jnp.zeros((pad,), jnp.float32)]
                          ).reshape(tot_ch, K)

    deg_parts = _make_deg_call(epw)(
        dst3.reshape(NW, epw), ew3.reshape(NW, epw))

    dis_b = pl.pallas_call(
        _dis_body,
        grid=(N_PAD // _BLKN,),
        in_specs=[pl.BlockSpec((NW, _BLKN), lambda i: (0, i))],
        out_specs=_row_spec(_BLKN),
        out_shape=jax.ShapeDtypeStruct((N_PAD, D), jnp.float32),
    )(deg_parts)

    h1p = pl.pallas_call(
        _h1_body,
        grid=(n // _R,),
        in_specs=[_row_spec(_R), _full_spec((D, D)), _row_spec(_R)],
        out_specs=_row_spec(_R),
        out_shape=jax.ShapeDtypeStruct((n, D), jnp.float32),
    )(x, W1, dis_b)

    agg_call = _make_agg_call(n, _CH_SPLIT)
    agg1 = agg_call(h1p, src3, dst3, ew3)

    h2p = pl.pallas_call(
        _mid_body,
        grid=(n // _R,),
        in_specs=[_row_spec(_R), _row_spec(_R), _row_spec(_R), _row_spec(_R),
                  _full_spec((1, D)), _full_spec((D, D))],
        out_specs=_row_spec(_R),
        out_shape=jax.ShapeDtypeStruct((n, D), jnp.float32),
    )(agg1[0], agg1[1], h1p, dis_b, b1.reshape(1, D), W2)

    agg2 = agg_call(h2p, src3, dst3, ew3)

    out = pl.pallas_call(
        _out_body,
        grid=(n // _R,),
        in_specs=[_row_spec(_R), _row_spec(_R), _row_spec(_R), _row_spec(_R),
                  _full_spec((1, D))],
        out_specs=_row_spec(_R),
        out_shape=jax.ShapeDtypeStruct((n, D), jnp.float32),
    )(agg2[0], agg2[1], h2p, dis_b, b2.reshape(1, D))
    return out


# speedup vs baseline: 2.7171x; 2.5652x over previous
import functools

import jax
import jax.numpy as jnp
from jax import lax
from jax.experimental import pallas as pl
from jax.experimental.pallas import tpu as pltpu
from jax.experimental.pallas import tpu_sc as plsc

N_NODES = 10000
D = 128
NC, NS, LANES = 2, 16, 16
NW = NC * NS
K = 128
NBUF = 2

_mesh = plsc.VectorSubcoreMesh(
    core_axis_name="c", subcore_axis_name="s", num_cores=NC, num_subcores=NS)


N_PAD = 10240


def _deg_body(dst_hbm, ew_hbm, out_hbm, dst_v, ew_v, deg_v):
    epw = dst_v.shape[0]
    c = lax.axis_index("c")
    s = lax.axis_index("s")
    wid = c * NS + s
    zeros = jnp.zeros((LANES,), jnp.float32)

    def zbody(i, carry):
        deg_v[pl.ds(i * LANES, LANES)] = zeros
        return carry

    lax.fori_loop(0, N_PAD // LANES, zbody, 0)
    pltpu.sync_copy(dst_hbm.at[wid], dst_v)
    pltpu.sync_copy(ew_hbm.at[wid], ew_v)

    def body(g, carry):
        idx = dst_v[pl.ds(g * LANES, LANES)]
        val = ew_v[pl.ds(g * LANES, LANES)]
        plsc.addupdate_scatter(deg_v, [idx], val)
        return carry

    lax.fori_loop(0, epw // LANES, body, 0)
    pltpu.sync_copy(deg_v, out_hbm.at[wid])


def _make_deg_call(epw):
    return pl.kernel(
        _deg_body,
        out_type=jax.ShapeDtypeStruct((NW, N_PAD), jnp.float32),
        mesh=_mesh,
        scratch_types=[
            pltpu.VMEM((epw,), jnp.int32),
            pltpu.VMEM((epw,), jnp.float32),
            pltpu.VMEM((N_PAD,), jnp.float32),
        ],
        compiler_params=pltpu.CompilerParams(needs_layout_passes=False),
    )


def _agg_body(ch_split, hp, src2, dst2, ew2, out_hbm, sidx_v, didx_v, ew_v,
              rows_v, acc_sh, gsem, ssem):
    ch0, ch1 = ch_split
    n_acc = out_hbm.shape[1]
    c = lax.axis_index("c")
    s = lax.axis_index("s")
    ch_n = jnp.where(c == 0, ch0, ch1)
    base = jnp.where(c == 0, s * ch0, NS * ch0 + s * ch1)
    rpt = 632

    def start_chunk(ch, b):
        row = base + ch
        pltpu.sync_copy(src2.at[row], sidx_v.at[pl.ds(b * K, K)])
        pltpu.sync_copy(dst2.at[row], didx_v.at[b])
        pltpu.sync_copy(ew2.at[row], ew_v.at[pl.ds(b * K, K)])
        pltpu.async_copy(hp.at[sidx_v.at[pl.ds(b * K, K)]], rows_v.at[b],
                         gsem.at[b])

    def gather_wait(b):
        pltpu.make_async_copy(hp.at[sidx_v.at[pl.ds(b * K, K)]],
                              rows_v.at[b], gsem.at[b]).wait()

    def scatter_start(b):
        pltpu.async_copy(rows_v.at[b], acc_sh.at[didx_v.at[b]],
                         ssem.at[b], add=True)

    def scatter_wait(b):
        pltpu.make_async_copy(rows_v.at[b], acc_sh.at[didx_v.at[b]],
                              ssem.at[b]).wait()

    zeros = jnp.zeros((LANES,), jnp.float32)

    def zbody(i, carry):
        for v in range(D // LANES):
            rows_v[0, i, pl.ds(v * LANES, LANES)] = zeros
        return carry

    lax.fori_loop(0, K, zbody, 0)

    _blk = [(0, K), (K, K), (2 * K, K), (3 * K, K), (4 * K, rpt - 4 * K)]

    def acc_off(boff, sz):
        off = jnp.minimum(s * rpt + boff, n_acc - sz)
        return pl.multiple_of(off, 8)

    for boff, sz in _blk:
        pltpu.sync_copy(rows_v.at[0, pl.ds(0, sz)],
                        acc_sh.at[pl.ds(acc_off(boff, sz), sz)])
    plsc.subcore_barrier()

    start_chunk(0, 0)
    start_chunk(1, 1)

    def process(ch, b):
        gather_wait(b)

        def wgroup(g, carry2, _b=b):
            ewv = ew_v[pl.ds(_b * K + g * LANES, LANES)]
            for l in range(LANES):
                w = lax.gather(
                    ewv, jnp.full((LANES, 1), l, jnp.int32),
                    lax.GatherDimensionNumbers(
                        offset_dims=(), collapsed_slice_dims=(0,),
                        start_index_map=(0,)),
                    slice_sizes=(1,),
                    mode=lax.GatherScatterMode.PROMISE_IN_BOUNDS)
                r = g * LANES + l
                for v in range(D // LANES):
                    sli = pl.ds(v * LANES, LANES)
                    rows_v[_b, r, sli] = rows_v[_b, r, sli] * w
            return carry2

        lax.fori_loop(0, K // LANES, wgroup, 0)
        scatter_start(b)

        def nxt():
            scatter_wait(b)
            start_chunk(ch + NBUF, b)

        pl.when(ch + NBUF < ch_n)(nxt)

    def outer(o, carry):
        for b in range(NBUF):
            process(o * NBUF + b, b)
        return carry

    lax.fori_loop(0, ch_n // NBUF, outer, 0)
    for b in range(NBUF):
        scatter_wait(b)
    plsc.subcore_barrier()

    for boff, sz in _blk:
        off = acc_off(boff, sz)
        pltpu.sync_copy(acc_sh.at[pl.ds(off, sz)],
                        rows_v.at[0, pl.ds(0, sz)])
        pltpu.sync_copy(rows_v.at[0, pl.ds(0, sz)],
                        out_hbm.at[c, pl.ds(off, sz)])


def _make_agg_call(n, ch_split):
    return pl.kernel(
        functools.partial(_agg_body, ch_split),
        out_type=jax.ShapeDtypeStruct((NC, n, D), jnp.float32),
        mesh=_mesh,
        scratch_types=[
            pltpu.VMEM((NBUF * K,), jnp.int32),
            pltpu.VMEM((NBUF, K), jnp.int32),
            pltpu.VMEM((NBUF * K,), jnp.float32),
            pltpu.VMEM((NBUF, K, D), jnp.float32),
            pltpu.VMEM_SHARED((n, D), jnp.float32),
            pltpu.SemaphoreType.DMA((NBUF,)),
            pltpu.SemaphoreType.DMA((NBUF,)),
        ],
        compiler_params=pltpu.CompilerParams(needs_layout_passes=False),
    )


_BLKN = 2048
_R = 2000


def _dis_body(parts_ref, dis_ref):
    deg = jnp.sum(parts_ref[...], axis=0) + 1.0
    dis = lax.rsqrt(deg)
    dis_ref[...] = jnp.broadcast_to(dis[:, None], (_BLKN, D))


def _h1_body(x_ref, w_ref, dis_ref, out_ref):
    h = jnp.dot(x_ref[...], w_ref[...], preferred_element_type=jnp.float32)
    out_ref[...] = h * dis_ref[...]


def _mid_body(a0_ref, a1_ref, hp_ref, dis_ref, b1_ref, w2_ref, out_ref):
    t = (a0_ref[...] + a1_ref[...] + hp_ref[...]) * dis_ref[...] + b1_ref[...]
    t = jnp.maximum(t, 0.0)
    h = jnp.dot(t, w2_ref[...], preferred_element_type=jnp.float32)
    out_ref[...] = h * dis_ref[...]


def _out_body(a0_ref, a1_ref, hp_ref, dis_ref, b2_ref, out_ref):
    out_ref[...] = ((a0_ref[...] + a1_ref[...] + hp_ref[...]) * dis_ref[...]
                    + b2_ref[...])


def _row_spec(r):
    return pl.BlockSpec((r, D), lambda i: (i, 0))


def _full_spec(shape):
    return pl.BlockSpec(shape, lambda i: tuple(0 for _ in shape))


_CH_SPLIT = (112, 48)


@jax.jit
def kernel(x, edge_index, edge_attr, W1, b1, W2, b2):
    n, d = x.shape
    e = edge_index.shape[1]
    ch0, ch1 = _CH_SPLIT
    tot_ch = NS * (ch0 + ch1)
    e_pad = tot_ch * K
    assert e_pad >= e and min(ch0, ch1) >= 4
    epw = e_pad // NW

    src = edge_index[0].astype(jnp.int32)
    dst = edge_index[1].astype(jnp.int32)
    ew = edge_attr.astype(jnp.float32)
    pad = e_pad - e
    src3 = jnp.concatenate([src, jnp.zeros((pad,), jnp.int32)]
                           ).reshape(tot_ch, K)
    dst3 = jnp.concatenate([dst, jnp.zeros((pad,), jnp.int32)]
                           ).reshape(tot_ch, K)
    ew3 = jnp.concatenate([ew, jnp.zeros((pad,), jnp.float32)]
                          ).reshape(tot_ch, K)

    deg_parts = _make_deg_call(epw)(
        dst3.reshape(NW, epw), ew3.reshape(NW, epw))

    dis_b = pl.pallas_call(
        _dis_body,
        grid=(N_PAD // _BLKN,),
        in_specs=[pl.BlockSpec((NW, _BLKN), lambda i: (0, i))],
        out_specs=_row_spec(_BLKN),
        out_shape=jax.ShapeDtypeStruct((N_PAD, D), jnp.float32),
    )(deg_parts)

    h1p = pl.pallas_call(
        _h1_body,
        grid=(n // _R,),
        in_specs=[_row_spec(_R), _full_spec((D, D)), _row_spec(_R)],
        out_specs=_row_spec(_R),
        out_shape=jax.ShapeDtypeStruct((n, D), jnp.float32),
    )(x, W1, dis_b)

    agg_call = _make_agg_call(n, _CH_SPLIT)
    agg1 = agg_call(h1p, src3, dst3, ew3)

    h2p = pl.pallas_call(
        _mid_body,
        grid=(n // _R,),
        in_specs=[_row_spec(_R), _row_spec(_R), _row_spec(_R), _row_spec(_R),
                  _full_spec((1, D)), _full_spec((D, D))],
        out_specs=_row_spec(_R),
        out_shape=jax.ShapeDtypeStruct((n, D), jnp.float32),
    )(agg1[0], agg1[1], h1p, dis_b, b1.reshape(1, D), W2)

    agg2 = agg_call(h2p, src3, dst3, ew3)

    out = pl.pallas_call(
        _out_body,
        grid=(n // _R,),
        in_specs=[_row_spec(_R), _row_spec(_R), _row_spec(_R), _row_spec(_R),
                  _full_spec((1, D))],
        out_specs=_row_spec(_R),
        out_shape=jax.ShapeDtypeStruct((n, D), jnp.float32),
    )(agg2[0], agg2[1], h2p, dis_b, b2.reshape(1, D))
    return out


# speedup vs baseline: 3.1838x; 1.1718x over previous
import functools

import jax
import jax.numpy as jnp
from jax import lax
from jax.experimental import pallas as pl
from jax.experimental.pallas import tpu as pltpu
from jax.experimental.pallas import tpu_sc as plsc

N_NODES = 10000
D = 128
NC, NS, LANES = 2, 16, 16
NW = NC * NS
K = 128
NBUF = 2

_mesh = plsc.VectorSubcoreMesh(
    core_axis_name="c", subcore_axis_name="s", num_cores=NC, num_subcores=NS)


N_PAD = 10240


def _deg_body(dst_hbm, ew_hbm, out_hbm, dst_v, ew_v, deg_v):
    epw = dst_v.shape[0]
    c = lax.axis_index("c")
    s = lax.axis_index("s")
    wid = c * NS + s
    zeros = jnp.zeros((LANES,), jnp.float32)

    def zbody(i, carry):
        deg_v[pl.ds(i * LANES, LANES)] = zeros
        return carry

    lax.fori_loop(0, N_PAD // LANES, zbody, 0)
    pltpu.sync_copy(dst_hbm.at[wid], dst_v)
    pltpu.sync_copy(ew_hbm.at[wid], ew_v)

    def body(g, carry):
        idx = dst_v[pl.ds(g * LANES, LANES)]
        val = ew_v[pl.ds(g * LANES, LANES)]
        plsc.addupdate_scatter(deg_v, [idx], val)
        return carry

    lax.fori_loop(0, epw // LANES, body, 0)
    pltpu.sync_copy(deg_v, out_hbm.at[wid])


def _make_deg_call(epw):
    return pl.kernel(
        _deg_body,
        out_type=jax.ShapeDtypeStruct((NW, N_PAD), jnp.float32),
        mesh=_mesh,
        scratch_types=[
            pltpu.VMEM((epw,), jnp.int32),
            pltpu.VMEM((epw,), jnp.float32),
            pltpu.VMEM((N_PAD,), jnp.float32),
        ],
        compiler_params=pltpu.CompilerParams(needs_layout_passes=False),
    )


def _agg_body(ch_split, hp, src2, dst2, ew2, out_hbm, sidx_v, didx_v, ew_v,
              rows_v, acc_sh, gsem, ssem):
    ch0, ch1 = ch_split
    n_acc = out_hbm.shape[1]
    c = lax.axis_index("c")
    s = lax.axis_index("s")
    ch_n = jnp.where(c == 0, ch0, ch1)
    base = jnp.where(c == 0, s * ch0, NS * ch0 + s * ch1)
    rpt = 632

    def start_chunk(ch, b):
        row = base + ch
        pltpu.sync_copy(src2.at[row], sidx_v.at[pl.ds(b * K, K)])
        pltpu.sync_copy(dst2.at[row], didx_v.at[b])
        pltpu.sync_copy(ew2.at[row], ew_v.at[pl.ds(b * K, K)])
        pltpu.async_copy(hp.at[sidx_v.at[pl.ds(b * K, K)]], rows_v.at[b],
                         gsem.at[b])

    def gather_wait(b):
        pltpu.make_async_copy(hp.at[sidx_v.at[pl.ds(b * K, K)]],
                              rows_v.at[b], gsem.at[b]).wait()

    def scatter_start(b):
        pltpu.async_copy(rows_v.at[b], acc_sh.at[didx_v.at[b]],
                         ssem.at[b], add=True)

    def scatter_wait(b):
        pltpu.make_async_copy(rows_v.at[b], acc_sh.at[didx_v.at[b]],
                              ssem.at[b]).wait()

    zeros = jnp.zeros((LANES,), jnp.float32)

    def zbody(i, carry):
        for v in range(D // LANES):
            rows_v[0, i, pl.ds(v * LANES, LANES)] = zeros
        return carry

    lax.fori_loop(0, K, zbody, 0)

    _blk = [(0, K), (K, K), (2 * K, K), (3 * K, K), (4 * K, rpt - 4 * K)]

    def acc_off(boff, sz):
        off = jnp.minimum(s * rpt + boff, n_acc - sz)
        return pl.multiple_of(off, 8)

    for boff, sz in _blk:
        pltpu.sync_copy(rows_v.at[0, pl.ds(0, sz)],
                        acc_sh.at[pl.ds(acc_off(boff, sz), sz)])
    plsc.subcore_barrier()

    start_chunk(0, 0)
    start_chunk(1, 1)

    def process(ch, b):
        gather_wait(b)

        def wgroup(g, carry2, _b=b):
            ewv = ew_v[pl.ds(_b * K + g * LANES, LANES)]
            for l in range(LANES):
                w = lax.gather(
                    ewv, jnp.full((LANES, 1), l, jnp.int32),
                    lax.GatherDimensionNumbers(
                        offset_dims=(), collapsed_slice_dims=(0,),
                        start_index_map=(0,)),
                    slice_sizes=(1,),
                    mode=lax.GatherScatterMode.PROMISE_IN_BOUNDS)
                r = g * LANES + l
                for v in range(D // LANES):
                    sli = pl.ds(v * LANES, LANES)
                    rows_v[_b, r, sli] = rows_v[_b, r, sli] * w
            return carry2

        lax.fori_loop(0, K // LANES, wgroup, 0)
        scatter_start(b)

        def nxt():
            scatter_wait(b)
            start_chunk(ch + NBUF, b)

        pl.when(ch + NBUF < ch_n)(nxt)

    def outer(o, carry):
        for b in range(NBUF):
            process(o * NBUF + b, b)
        return carry

    lax.fori_loop(0, ch_n // NBUF, outer, 0)
    for b in range(NBUF):
        scatter_wait(b)
    plsc.subcore_barrier()

    for boff, sz in _blk:
        off = acc_off(boff, sz)
        pltpu.sync_copy(acc_sh.at[pl.ds(off, sz)],
                        rows_v.at[0, pl.ds(0, sz)])
        pltpu.sync_copy(rows_v.at[0, pl.ds(0, sz)],
                        out_hbm.at[c, pl.ds(off, sz)])


def _make_agg_call(n, ch_split):
    return pl.kernel(
        functools.partial(_agg_body, ch_split),
        out_type=jax.ShapeDtypeStruct((NC, n, D), jnp.float32),
        mesh=_mesh,
        scratch_types=[
            pltpu.VMEM((NBUF * K,), jnp.int32),
            pltpu.VMEM((NBUF, K), jnp.int32),
            pltpu.VMEM((NBUF * K,), jnp.float32),
            pltpu.VMEM((NBUF, K, D), jnp.float32),
            pltpu.VMEM_SHARED((n, D), jnp.float32),
            pltpu.SemaphoreType.DMA((NBUF,)),
            pltpu.SemaphoreType.DMA((NBUF,)),
        ],
        compiler_params=pltpu.CompilerParams(needs_layout_passes=False),
    )


_BLKN = 2048
_R = 2000


def _dis_body(parts_ref, dis_ref):
    deg = jnp.sum(parts_ref[...], axis=0) + 1.0
    dis = lax.rsqrt(deg)
    dis_ref[...] = jnp.broadcast_to(dis[:, None], (_BLKN, D))


def _h1_body(x_ref, w_ref, dis_ref, out_ref):
    h = jnp.dot(x_ref[...], w_ref[...], preferred_element_type=jnp.float32)
    out_ref[...] = h * dis_ref[...]


def _mid_body(a0_ref, a1_ref, hp_ref, dis_ref, b1_ref, w2_ref, out_ref):
    t = (a0_ref[...] + a1_ref[...] + hp_ref[...]) * dis_ref[...] + b1_ref[...]
    t = jnp.maximum(t, 0.0)
    h = jnp.dot(t, w2_ref[...], preferred_element_type=jnp.float32)
    out_ref[...] = h * dis_ref[...]


def _out_body(a0_ref, a1_ref, hp_ref, dis_ref, b2_ref, out_ref):
    out_ref[...] = ((a0_ref[...] + a1_ref[...] + hp_ref[...]) * dis_ref[...]
                    + b2_ref[...])


def _row_spec(r):
    return pl.BlockSpec((r, D), lambda i: (i, 0))


def _full_spec(shape):
    return pl.BlockSpec(shape, lambda i: tuple(0 for _ in shape))


_CH_SPLIT = (118, 40)


@jax.jit
def kernel(x, edge_index, edge_attr, W1, b1, W2, b2):
    n, d = x.shape
    e = edge_index.shape[1]
    ch0, ch1 = _CH_SPLIT
    tot_ch = NS * (ch0 + ch1)
    e_pad = tot_ch * K
    assert e_pad >= e and min(ch0, ch1) >= 4
    epw = e_pad // NW

    src = edge_index[0].astype(jnp.int32)
    dst = edge_index[1].astype(jnp.int32)
    ew = edge_attr.astype(jnp.float32)
    pad = e_pad - e
    src3 = jnp.concatenate([src, jnp.zeros((pad,), jnp.int32)]
                           ).reshape(tot_ch, K)
    dst3 = jnp.concatenate([dst, jnp.zeros((pad,), jnp.int32)]
                           ).reshape(tot_ch, K)
    ew3 = jnp.concatenate([ew, jnp.zeros((pad,), jnp.float32)]
                          ).reshape(tot_ch, K)

    deg_parts = _make_deg_call(epw)(
        dst3.reshape(NW, epw), ew3.reshape(NW, epw))

    dis_b = pl.pallas_call(
        _dis_body,
        grid=(N_PAD // _BLKN,),
        in_specs=[pl.BlockSpec((NW, _BLKN), lambda i: (0, i))],
        out_specs=_row_spec(_BLKN),
        out_shape=jax.ShapeDtypeStruct((N_PAD, D), jnp.float32),
    )(deg_parts)

    h1p = pl.pallas_call(
        _h1_body,
        grid=(n // _R,),
        in_specs=[_row_spec(_R), _full_spec((D, D)), _row_spec(_R)],
        out_specs=_row_spec(_R),
        out_shape=jax.ShapeDtypeStruct((n, D), jnp.float32),
    )(x, W1, dis_b)

    agg_call = _make_agg_call(n, _CH_SPLIT)
    agg1 = agg_call(h1p, src3, dst3, ew3)

    h2p = pl.pallas_call(
        _mid_body,
        grid=(n // _R,),
        in_specs=[_row_spec(_R), _row_spec(_R), _row_spec(_R), _row_spec(_R),
                  _full_spec((1, D)), _full_spec((D, D))],
        out_specs=_row_spec(_R),
        out_shape=jax.ShapeDtypeStruct((n, D), jnp.float32),
    )(agg1[0], agg1[1], h1p, dis_b, b1.reshape(1, D), W2)

    agg2 = agg_call(h2p, src3, dst3, ew3)

    out = pl.pallas_call(
        _out_body,
        grid=(n // _R,),
        in_specs=[_row_spec(_R), _row_spec(_R), _row_spec(_R), _row_spec(_R),
                  _full_spec((1, D))],
        out_specs=_row_spec(_R),
        out_shape=jax.ShapeDtypeStruct((n, D), jnp.float32),
    )(agg2[0], agg2[1], h2p, dis_b, b2.reshape(1, D))
    return out
